# baseline (speedup 1.0000x reference)
import functools

import jax
import jax.numpy as jnp
from jax import lax
from jax.experimental import pallas as pl
from jax.experimental.pallas import tpu as pltpu
S, D, E, TOPK, FF = 2048, 768, 8, 2, 2048
TB = 256
NB = (S * TOPK) // TB + E
NPP = NB * TB
NW = 32
CH = NPP // NW
NPB = 2048
ND = NPP // NPB


def _gating_body(x_ref, wr_ref, pos1_ref, pos2_ref, w1_ref, w2_ref,
                 be_ref, loss_ref, xbf_ref):
    x = x_ref[...]
    wr = wr_ref[...]
    xbf_ref[...] = x.astype(jnp.bfloat16)
    logits = jax.lax.dot_general(
        x, wr, (((1,), (1,)), ((), ())),
        preferred_element_type=jnp.float32,
    )
    m = jnp.max(logits, axis=-1, keepdims=True)
    p = jnp.exp(logits - m)
    p = p / jnp.sum(p, axis=-1, keepdims=True)

    g1 = jnp.max(p, axis=-1, keepdims=True)
    i1 = jnp.argmax(p, axis=-1, keepdims=True)
    lanes = jax.lax.broadcasted_iota(jnp.int32, (S, E), 1)
    p2 = jnp.where(lanes == i1, -jnp.inf, p)
    g2 = jnp.max(p2, axis=-1, keepdims=True)
    i2 = jnp.argmax(p2, axis=-1, keepdims=True)
    denom = g1 + g2 + 1e-8
    w1_ref[...] = g1 / denom
    w2_ref[...] = g2 / denom

    sel1 = (lanes == i1).astype(jnp.float32)
    sel2 = (lanes == i2).astype(jnp.float32)
    cnt1 = jnp.sum(sel1, axis=0, keepdims=True)
    cnt2 = jnp.sum(sel2, axis=0, keepdims=True)
    counts = cnt1 + cnt2
    usage = counts / jnp.sum(counts)
    mean = jnp.mean(usage)
    var = jnp.sum((usage - mean) ** 2) / (E - 1)
    cv2 = (var / (mean + 1e-8)) ** 2
    loss_ref[0, 0] = cv2

    ti = jax.lax.broadcasted_iota(jnp.int32, (S, S), 0)
    tj = jax.lax.broadcasted_iota(jnp.int32, (S, S), 1)
    ltri = (ti > tj).astype(jnp.bfloat16)
    sel12 = jnp.concatenate([sel1, sel2], axis=1).astype(jnp.bfloat16)
    cums = jax.lax.dot_general(
        ltri, sel12, (((1,), (0,)), ((), ())),
        preferred_element_type=jnp.float32)
    excl1 = cums[:, :E]
    excl2 = cums[:, E:]
    pcnt = jnp.floor((counts + (TB - 1)) / TB) * TB
    ui = jax.lax.broadcasted_iota(jnp.int32, (E, E), 0)
    uj = jax.lax.broadcasted_iota(jnp.int32, (E, E), 1)
    utri = (ui <= uj).astype(jnp.float32)
    pincl = jax.lax.dot_general(
        pcnt, utri, (((1,), (0,)), ((), ())),
        preferred_element_type=jnp.float32)
    poff = pincl - pcnt
    pos1 = jnp.sum(sel1 * (poff + excl1), axis=1, keepdims=True)
    pos2 = jnp.sum(sel2 * (poff + cnt1 + excl2), axis=1, keepdims=True)
    pos1_ref[...] = pos1.astype(jnp.int32)
    pos2_ref[...] = pos2.astype(jnp.int32)

    bend = pincl.astype(jnp.int32) // TB
    biota = jax.lax.broadcasted_iota(jnp.int32, (NB, E), 0)
    be = jnp.sum((biota >= bend).astype(jnp.int32), axis=1, keepdims=True)
    be_ref[...] = jnp.minimum(be, E - 1)


def _ffn_body(ea_ref, pos1_ref, pos2_ref, xbf_ref, wg_ref, wu_ref, wd_ref,
              y_ref):
    i = pl.program_id(0)
    riota = jax.lax.broadcasted_iota(jnp.int32, (S, TB), 1) + i * TB
    sel = ((riota == pos1_ref[...]) | (riota == pos2_ref[...])
           ).astype(jnp.bfloat16)
    x = jax.lax.dot_general(
        sel, xbf_ref[...], (((0,), (0,)), ((), ())),
        preferred_element_type=jnp.float32)
    g = jax.lax.dot_general(
        x, wg_ref[0], (((1,), (1,)), ((), ())),
        preferred_element_type=jnp.float32)
    u = jax.lax.dot_general(
        x, wu_ref[0], (((1,), (1,)), ((), ())),
        preferred_element_type=jnp.float32)
    act = g * (u * jax.nn.sigmoid(u))
    y = jax.lax.dot_general(
        act, wd_ref[0], (((1,), (1,)), ((), ())),
        preferred_element_type=jnp.float32)
    y_ref[...] = y


@jax.jit
def kernel(x, Wg, Wu, Wd, Wr):
    b, s, d = x.shape
    x2 = x.reshape(s, d)

    pos1, pos2, w1, w2, be, loss, x_bf = pl.pallas_call(
        _gating_body,
        out_shape=(
            jax.ShapeDtypeStruct((S, 1), jnp.int32),
            jax.ShapeDtypeStruct((S, 1), jnp.int32),
            jax.ShapeDtypeStruct((S, 1), jnp.float32),
            jax.ShapeDtypeStruct((S, 1), jnp.float32),
            jax.ShapeDtypeStruct((NB, 1), jnp.int32),
            jax.ShapeDtypeStruct((1, 1), jnp.float32),
            jax.ShapeDtypeStruct((S, D), jnp.bfloat16),
        ),
        in_specs=[
            pl.BlockSpec((S, D), lambda: (0, 0)),
            pl.BlockSpec((E, D), lambda: (0, 0)),
        ],
        out_specs=(
            pl.BlockSpec((S, 1), lambda: (0, 0)),
            pl.BlockSpec((S, 1), lambda: (0, 0)),
            pl.BlockSpec((S, 1), lambda: (0, 0)),
            pl.BlockSpec((S, 1), lambda: (0, 0)),
            pl.BlockSpec((NB, 1), lambda: (0, 0)),
            pl.BlockSpec(memory_space=pltpu.SMEM),
            pl.BlockSpec((S, D), lambda: (0, 0)),
        ),
    )(x2, Wr)


    y = pl.pallas_call(
        _ffn_body,
        grid_spec=pltpu.PrefetchScalarGridSpec(
            num_scalar_prefetch=1,
            grid=(NB,),
            in_specs=[
                pl.BlockSpec((S, 1), lambda i, ea: (0, 0)),
                pl.BlockSpec((S, 1), lambda i, ea: (0, 0)),
                pl.BlockSpec((S, D), lambda i, ea: (0, 0)),
                pl.BlockSpec((1, FF, D), lambda i, ea: (ea[i], 0, 0)),
                pl.BlockSpec((1, FF, D), lambda i, ea: (ea[i], 0, 0)),
                pl.BlockSpec((1, D, FF), lambda i, ea: (ea[i], 0, 0)),
            ],
            out_specs=pl.BlockSpec((TB, D), lambda i, ea: (i, 0)),
        ),
        out_shape=jax.ShapeDtypeStruct((NPP, D), jnp.float32),
    )(be.reshape(NB), pos1, pos2, x_bf, Wg, Wu, Wd)

    out = w1 * y[pos1[:, 0]] + w2 * y[pos2[:, 0]]
    return out.reshape(b, s, d), loss.reshape(())


# speedup vs baseline: 1.0702x; 1.0702x over previous
import functools

import jax
import jax.numpy as jnp
from jax import lax
from jax.experimental import pallas as pl
from jax.experimental.pallas import tpu as pltpu
S, D, E, TOPK, FF = 2048, 768, 8, 2, 2048
TB = 256
NB = (S * TOPK) // TB + E
NPP = NB * TB
NW = 32
CH = NPP // NW
NPB = 2048
ND = NPP // NPB


def _gating_body(x_ref, wr_ref, pos1_ref, pos2_ref, w1_ref, w2_ref,
                 be_ref, loss_ref, xbf_ref):
    x = x_ref[...]
    wr = wr_ref[...]
    xbf_ref[...] = x.astype(jnp.bfloat16)
    logits = jax.lax.dot_general(
        x, wr, (((1,), (1,)), ((), ())),
        preferred_element_type=jnp.float32,
    )
    m = jnp.max(logits, axis=-1, keepdims=True)
    p = jnp.exp(logits - m)
    p = p / jnp.sum(p, axis=-1, keepdims=True)

    g1 = jnp.max(p, axis=-1, keepdims=True)
    i1 = jnp.argmax(p, axis=-1, keepdims=True)
    lanes = jax.lax.broadcasted_iota(jnp.int32, (S, E), 1)
    p2 = jnp.where(lanes == i1, -jnp.inf, p)
    g2 = jnp.max(p2, axis=-1, keepdims=True)
    i2 = jnp.argmax(p2, axis=-1, keepdims=True)
    denom = g1 + g2 + 1e-8
    w1_ref[...] = g1 / denom
    w2_ref[...] = g2 / denom

    sel1 = (lanes == i1).astype(jnp.float32)
    sel2 = (lanes == i2).astype(jnp.float32)
    cnt1 = jnp.sum(sel1, axis=0, keepdims=True)
    cnt2 = jnp.sum(sel2, axis=0, keepdims=True)
    counts = cnt1 + cnt2
    usage = counts / jnp.sum(counts)
    mean = jnp.mean(usage)
    var = jnp.sum((usage - mean) ** 2) / (E - 1)
    cv2 = (var / (mean + 1e-8)) ** 2
    loss_ref[0, 0] = cv2

    ti = jax.lax.broadcasted_iota(jnp.int32, (S, S), 0)
    tj = jax.lax.broadcasted_iota(jnp.int32, (S, S), 1)
    ltri = (ti > tj).astype(jnp.bfloat16)
    sel12 = jnp.concatenate([sel1, sel2], axis=1).astype(jnp.bfloat16)
    cums = jax.lax.dot_general(
        ltri, sel12, (((1,), (0,)), ((), ())),
        preferred_element_type=jnp.float32)
    excl1 = cums[:, :E]
    excl2 = cums[:, E:]
    pcnt = jnp.floor((counts + (TB - 1)) / TB) * TB
    ui = jax.lax.broadcasted_iota(jnp.int32, (E, E), 0)
    uj = jax.lax.broadcasted_iota(jnp.int32, (E, E), 1)
    utri = (ui <= uj).astype(jnp.float32)
    pincl = jax.lax.dot_general(
        pcnt, utri, (((1,), (0,)), ((), ())),
        preferred_element_type=jnp.float32)
    poff = pincl - pcnt
    pos1 = jnp.sum(sel1 * (poff + excl1), axis=1, keepdims=True)
    pos2 = jnp.sum(sel2 * (poff + cnt1 + excl2), axis=1, keepdims=True)
    pos1_ref[...] = pos1.astype(jnp.int32)
    pos2_ref[...] = pos2.astype(jnp.int32)

    bend = pincl.astype(jnp.int32) // TB
    biota = jax.lax.broadcasted_iota(jnp.int32, (NB, E), 0)
    be = jnp.sum((biota >= bend).astype(jnp.int32), axis=1, keepdims=True)
    be_ref[...] = jnp.minimum(be, E - 1)


def _dispatch_body(pos1_ref, pos2_ref, xbf_ref, xs_ref):
    i = pl.program_id(0)
    riota = jax.lax.broadcasted_iota(jnp.int32, (S, NPB), 1) + i * NPB
    p1 = pos1_ref[...]
    p2 = pos2_ref[...]
    sel = ((riota == p1) | (riota == p2)).astype(jnp.bfloat16)
    xs_ref[...] = jax.lax.dot_general(
        sel, xbf_ref[...], (((0,), (0,)), ((), ())),
        preferred_element_type=jnp.float32).astype(jnp.bfloat16)


def _ffn_body(ea_ref, xs_ref, wg_ref, wu_ref, wd_ref, y_ref):
    x = xs_ref[...]
    g = jax.lax.dot_general(
        x, wg_ref[0], (((1,), (1,)), ((), ())),
        preferred_element_type=jnp.float32)
    u = jax.lax.dot_general(
        x, wu_ref[0], (((1,), (1,)), ((), ())),
        preferred_element_type=jnp.float32)
    act = g * (u * jax.nn.sigmoid(u))
    y = jax.lax.dot_general(
        act, wd_ref[0], (((1,), (1,)), ((), ())),
        preferred_element_type=jnp.float32)
    y_ref[...] = y


@jax.jit
def kernel(x, Wg, Wu, Wd, Wr):
    b, s, d = x.shape
    x2 = x.reshape(s, d)

    pos1, pos2, w1, w2, be, loss, x_bf = pl.pallas_call(
        _gating_body,
        out_shape=(
            jax.ShapeDtypeStruct((S, 1), jnp.int32),
            jax.ShapeDtypeStruct((S, 1), jnp.int32),
            jax.ShapeDtypeStruct((S, 1), jnp.float32),
            jax.ShapeDtypeStruct((S, 1), jnp.float32),
            jax.ShapeDtypeStruct((NB, 1), jnp.int32),
            jax.ShapeDtypeStruct((1, 1), jnp.float32),
            jax.ShapeDtypeStruct((S, D), jnp.bfloat16),
        ),
        in_specs=[
            pl.BlockSpec((S, D), lambda: (0, 0)),
            pl.BlockSpec((E, D), lambda: (0, 0)),
        ],
        out_specs=(
            pl.BlockSpec((S, 1), lambda: (0, 0)),
            pl.BlockSpec((S, 1), lambda: (0, 0)),
            pl.BlockSpec((S, 1), lambda: (0, 0)),
            pl.BlockSpec((S, 1), lambda: (0, 0)),
            pl.BlockSpec((NB, 1), lambda: (0, 0)),
            pl.BlockSpec(memory_space=pltpu.SMEM),
            pl.BlockSpec((S, D), lambda: (0, 0)),
        ),
    )(x2, Wr)

    xs = pl.pallas_call(
        _dispatch_body,
        grid=(ND,),
        out_shape=jax.ShapeDtypeStruct((NPP, D), jnp.bfloat16),
        in_specs=[
            pl.BlockSpec((S, 1), lambda i: (0, 0)),
            pl.BlockSpec((S, 1), lambda i: (0, 0)),
            pl.BlockSpec((S, D), lambda i: (0, 0)),
        ],
        out_specs=pl.BlockSpec((NPB, D), lambda i: (i, 0)),
    )(pos1, pos2, x_bf)

    y = pl.pallas_call(
        _ffn_body,
        grid_spec=pltpu.PrefetchScalarGridSpec(
            num_scalar_prefetch=1,
            grid=(NB,),
            in_specs=[
                pl.BlockSpec((TB, D), lambda i, ea: (i, 0)),
                pl.BlockSpec((1, FF, D), lambda i, ea: (ea[i], 0, 0)),
                pl.BlockSpec((1, FF, D), lambda i, ea: (ea[i], 0, 0)),
                pl.BlockSpec((1, D, FF), lambda i, ea: (ea[i], 0, 0)),
            ],
            out_specs=pl.BlockSpec((TB, D), lambda i, ea: (i, 0)),
        ),
        out_shape=jax.ShapeDtypeStruct((NPP, D), jnp.float32),
    )(be.reshape(NB), xs, Wg, Wu, Wd)

    out = w1 * y[pos1[:, 0]] + w2 * y[pos2[:, 0]]
    return out.reshape(b, s, d), loss.reshape(())


# speedup vs baseline: 1.0959x; 1.0241x over previous
import functools

import jax
import jax.numpy as jnp
from jax import lax
from jax.experimental import pallas as pl
from jax.experimental.pallas import tpu as pltpu
S, D, E, TOPK, FF = 2048, 768, 8, 2, 2048
TB = 256
NB = (S * TOPK) // TB + E
NPP = NB * TB
NW = 32
CH = NPP // NW
NPB = 2048
ND = NPP // NPB


def _gating_body(x_ref, wr_ref, pos1_ref, pos2_ref, w1_ref, w2_ref,
                 be_ref, loss_ref, xbf_ref):
    x = x_ref[...]
    wr = wr_ref[...]
    xbf_ref[...] = x.astype(jnp.bfloat16)
    logits = jax.lax.dot_general(
        x, wr, (((1,), (1,)), ((), ())),
        preferred_element_type=jnp.float32,
    )
    m = jnp.max(logits, axis=-1, keepdims=True)
    p = jnp.exp(logits - m)
    p = p / jnp.sum(p, axis=-1, keepdims=True)

    g1 = jnp.max(p, axis=-1, keepdims=True)
    i1 = jnp.argmax(p, axis=-1, keepdims=True)
    lanes = jax.lax.broadcasted_iota(jnp.int32, (S, E), 1)
    p2 = jnp.where(lanes == i1, -jnp.inf, p)
    g2 = jnp.max(p2, axis=-1, keepdims=True)
    i2 = jnp.argmax(p2, axis=-1, keepdims=True)
    denom = g1 + g2 + 1e-8
    w1_ref[...] = g1 / denom
    w2_ref[...] = g2 / denom

    sel1 = (lanes == i1).astype(jnp.float32)
    sel2 = (lanes == i2).astype(jnp.float32)
    cnt1 = jnp.sum(sel1, axis=0, keepdims=True)
    cnt2 = jnp.sum(sel2, axis=0, keepdims=True)
    counts = cnt1 + cnt2
    usage = counts / jnp.sum(counts)
    mean = jnp.mean(usage)
    var = jnp.sum((usage - mean) ** 2) / (E - 1)
    cv2 = (var / (mean + 1e-8)) ** 2
    loss_ref[0, 0] = cv2

    ti = jax.lax.broadcasted_iota(jnp.int32, (S, S), 0)
    tj = jax.lax.broadcasted_iota(jnp.int32, (S, S), 1)
    ltri = (ti > tj).astype(jnp.bfloat16)
    sel12 = jnp.concatenate([sel1, sel2], axis=1).astype(jnp.bfloat16)
    cums = jax.lax.dot_general(
        ltri, sel12, (((1,), (0,)), ((), ())),
        preferred_element_type=jnp.float32)
    excl1 = cums[:, :E]
    excl2 = cums[:, E:]
    pcnt = jnp.floor((counts + (TB - 1)) / TB) * TB
    ui = jax.lax.broadcasted_iota(jnp.int32, (E, E), 0)
    uj = jax.lax.broadcasted_iota(jnp.int32, (E, E), 1)
    utri = (ui <= uj).astype(jnp.float32)
    pincl = jax.lax.dot_general(
        pcnt, utri, (((1,), (0,)), ((), ())),
        preferred_element_type=jnp.float32)
    poff = pincl - pcnt
    pos1 = jnp.sum(sel1 * (poff + excl1), axis=1, keepdims=True)
    pos2 = jnp.sum(sel2 * (poff + cnt1 + excl2), axis=1, keepdims=True)
    pos1_ref[...] = pos1.astype(jnp.int32)
    pos2_ref[...] = pos2.astype(jnp.int32)

    bend = pincl.astype(jnp.int32) // TB
    biota = jax.lax.broadcasted_iota(jnp.int32, (NB, E), 0)
    be = jnp.sum((biota >= bend).astype(jnp.int32), axis=1, keepdims=True)
    be_ref[...] = jnp.minimum(be, E - 1)


def _dispatch_body(pos1_ref, pos2_ref, xbf_ref, xs_ref):
    i = pl.program_id(0)
    riota = jax.lax.broadcasted_iota(jnp.int32, (S, NPB), 1) + i * NPB
    p1 = pos1_ref[...]
    p2 = pos2_ref[...]
    sel = ((riota == p1) | (riota == p2)).astype(jnp.bfloat16)
    xs_ref[...] = jax.lax.dot_general(
        sel, xbf_ref[...], (((0,), (0,)), ((), ())),
        preferred_element_type=jnp.float32).astype(jnp.bfloat16)


def _ffn_body(ea_ref, xs_ref, wg_ref, wu_ref, wd_ref, y_ref):
    x = xs_ref[...]
    g = jax.lax.dot_general(
        x, wg_ref[0], (((1,), (1,)), ((), ())),
        preferred_element_type=jnp.float32)
    u = jax.lax.dot_general(
        x, wu_ref[0], (((1,), (1,)), ((), ())),
        preferred_element_type=jnp.float32)
    act = g * (u * jax.nn.sigmoid(u))
    y = jax.lax.dot_general(
        act, wd_ref[0], (((1,), (1,)), ((), ())),
        preferred_element_type=jnp.float32)
    y_ref[...] = y.astype(jnp.bfloat16)


@jax.jit
def kernel(x, Wg, Wu, Wd, Wr):
    b, s, d = x.shape
    x2 = x.reshape(s, d)

    pos1, pos2, w1, w2, be, loss, x_bf = pl.pallas_call(
        _gating_body,
        out_shape=(
            jax.ShapeDtypeStruct((S, 1), jnp.int32),
            jax.ShapeDtypeStruct((S, 1), jnp.int32),
            jax.ShapeDtypeStruct((S, 1), jnp.float32),
            jax.ShapeDtypeStruct((S, 1), jnp.float32),
            jax.ShapeDtypeStruct((NB, 1), jnp.int32),
            jax.ShapeDtypeStruct((1, 1), jnp.float32),
            jax.ShapeDtypeStruct((S, D), jnp.bfloat16),
        ),
        in_specs=[
            pl.BlockSpec((S, D), lambda: (0, 0)),
            pl.BlockSpec((E, D), lambda: (0, 0)),
        ],
        out_specs=(
            pl.BlockSpec((S, 1), lambda: (0, 0)),
            pl.BlockSpec((S, 1), lambda: (0, 0)),
            pl.BlockSpec((S, 1), lambda: (0, 0)),
            pl.BlockSpec((S, 1), lambda: (0, 0)),
            pl.BlockSpec((NB, 1), lambda: (0, 0)),
            pl.BlockSpec(memory_space=pltpu.SMEM),
            pl.BlockSpec((S, D), lambda: (0, 0)),
        ),
    )(x2, Wr)

    xs = pl.pallas_call(
        _dispatch_body,
        grid=(ND,),
        out_shape=jax.ShapeDtypeStruct((NPP, D), jnp.bfloat16),
        in_specs=[
            pl.BlockSpec((S, 1), lambda i: (0, 0)),
            pl.BlockSpec((S, 1), lambda i: (0, 0)),
            pl.BlockSpec((S, D), lambda i: (0, 0)),
        ],
        out_specs=pl.BlockSpec((NPB, D), lambda i: (i, 0)),
    )(pos1, pos2, x_bf)

    y = pl.pallas_call(
        _ffn_body,
        grid_spec=pltpu.PrefetchScalarGridSpec(
            num_scalar_prefetch=1,
            grid=(NB,),
            in_specs=[
                pl.BlockSpec((TB, D), lambda i, ea: (i, 0)),
                pl.BlockSpec((1, FF, D), lambda i, ea: (ea[i], 0, 0)),
                pl.BlockSpec((1, FF, D), lambda i, ea: (ea[i], 0, 0)),
                pl.BlockSpec((1, D, FF), lambda i, ea: (ea[i], 0, 0)),
            ],
            out_specs=pl.BlockSpec((TB, D), lambda i, ea: (i, 0)),
        ),
        out_shape=jax.ShapeDtypeStruct((NPP, D), jnp.bfloat16),
    )(be.reshape(NB), xs, Wg, Wu, Wd)

    out = (w1 * y[pos1[:, 0]].astype(jnp.float32)
           + w2 * y[pos2[:, 0]].astype(jnp.float32))
    return out.reshape(b, s, d), loss.reshape(())
